# Initial kernel scaffold; baseline (speedup 1.0000x reference)
import jax
import jax.numpy as jnp
from jax.experimental import pallas as pl


def kernel(input, position_embeddings):
    raise NotImplementedError("write your pallas kernel here")


# speedup vs baseline: 1.5471x; 1.5471x over previous
import jax
import jax.numpy as jnp
from jax.experimental import pallas as pl


def kernel(input, position_embeddings):
    B, S, D = input.shape
    SBLK = 512

    def body(x_ref, pos_ref, o_ref):
        o_ref[...] = x_ref[...] + pos_ref[...][None]

    return pl.pallas_call(
        body,
        grid=(S // SBLK,),
        in_specs=[
            pl.BlockSpec((B, SBLK, D), lambda i: (0, i, 0)),
            pl.BlockSpec((SBLK, D), lambda i: (i, 0)),
        ],
        out_specs=pl.BlockSpec((B, SBLK, D), lambda i: (0, i, 0)),
        out_shape=jax.ShapeDtypeStruct((B, S, D), input.dtype),
    )(input, position_embeddings[:S])
